# Initial kernel scaffold; baseline (speedup 1.0000x reference)
import jax
import jax.numpy as jnp
from jax.experimental import pallas as pl


def kernel(x, edge_index, batch, Wl1, bl1, Wr1, Wl2, bl2, Wr2, W1, b1, W2, b2):
    raise NotImplementedError("write your pallas kernel here")


# speedup vs baseline: 3.0675x; 3.0675x over previous
import functools

import jax
import jax.numpy as jnp
from jax import lax
from jax.experimental import pallas as pl
from jax.experimental.pallas import tpu as pltpu
from jax.experimental.pallas import tpu_sc as plsc

N = 10000
E = 320000
D = 128
B = 16
OUT = 3

NC = 2
NS = 16
NW = NC * NS
CH = 128
GRP = 8
CPW = -(-E // (NW * CH * GRP)) * GRP
EPAD = NW * CH * CPW
NACC = -(-(N + 1) // (8 * NS)) * 8 * NS
RPS = NACC // NS


def _mesh():
    return plsc.VectorSubcoreMesh(core_axis_name="c", subcore_axis_name="s",
                                  num_cores=NC, num_subcores=NS)


def _zero_shared(buf_v, shared, base, width):
    def zrow(r, _):
        z = jnp.zeros((16,), jnp.float32)
        for l in range(width // 16):
            buf_v[r, pl.ds(l * 16, 16)] = z
        return 0
    lax.fori_loop(0, CH, zrow, 0)
    nfull = RPS // CH
    def zcopy(r, _):
        pltpu.sync_copy(buf_v, shared.at[pl.ds(base + r * CH, CH)])
        return 0
    lax.fori_loop(0, nfull, zcopy, 0)
    rem = RPS - nfull * CH
    if rem:
        pltpu.sync_copy(buf_v.at[pl.ds(0, rem)],
                        shared.at[pl.ds(base + nfull * CH, rem)])


def _sc_aggregate(x, srcg, dstg):
    scratch = [
        pltpu.VMEM_SHARED((NACC, D), jnp.float32),
        pltpu.VMEM((GRP, CH), jnp.int32),
        pltpu.VMEM((GRP, CH), jnp.int32),
        pltpu.VMEM((CH, D), jnp.float32),
        pltpu.SemaphoreType.DMA,
    ]

    def body(x_hbm, srcg_hbm, dstg_hbm, out_hbm, acc, src_v, dst_v, rows_v,
             sem):
        c = lax.axis_index("c")
        s = lax.axis_index("s")
        wid = s * NC + c
        base = s * RPS
        _zero_shared(rows_v, acc, base, D)
        plsc.subcore_barrier()

        def group(g, _):
            pltpu.sync_copy(srcg_hbm.at[wid].at[pl.ds(g * GRP, GRP)], src_v)
            pltpu.sync_copy(dstg_hbm.at[wid].at[pl.ds(g * GRP, GRP)], dst_v)
            for j in range(GRP):
                pltpu.async_copy(x_hbm.at[src_v.at[j]], rows_v, sem).wait()
                pltpu.sync_copy(rows_v, acc.at[dst_v.at[j]], add=True)
            return 0
        lax.fori_loop(0, CPW // GRP, group, 0)

        plsc.subcore_barrier()
        pltpu.sync_copy(acc.at[pl.ds(base, RPS)],
                        out_hbm.at[c].at[pl.ds(base, RPS)])

    k = pl.kernel(body, out_type=jax.ShapeDtypeStruct((NC, NACC, D),
                                                      jnp.float32),
                  mesh=_mesh(), scratch_types=scratch)
    return k(x, srcg, dstg)


def _sc_counts(dstg):
    scratch = [
        pltpu.VMEM_SHARED((NACC, D), jnp.float32),
        pltpu.VMEM((GRP, CH), jnp.int32),
        pltpu.VMEM((CH, D), jnp.float32),
    ]

    def body(dstg_hbm, cnt_hbm, cacc, dst_v, ones_v):
        c = lax.axis_index("c")
        s = lax.axis_index("s")
        wid = s * NC + c
        base = s * RPS
        _zero_shared(ones_v, cacc, base, D)
        def orow(r, _):
            one = jnp.ones((16,), jnp.float32)
            ones_v[r, pl.ds(0, 16)] = one
            return 0
        lax.fori_loop(0, CH, orow, 0)
        plsc.subcore_barrier()

        def group(g, _):
            pltpu.sync_copy(dstg_hbm.at[wid].at[pl.ds(g * GRP, GRP)], dst_v)
            for j in range(GRP):
                pltpu.sync_copy(ones_v, cacc.at[dst_v.at[j]], add=True)
            return 0
        lax.fori_loop(0, CPW // GRP, group, 0)

        plsc.subcore_barrier()
        pltpu.sync_copy(cacc.at[pl.ds(base, RPS)],
                        cnt_hbm.at[c].at[pl.ds(base, RPS)])

    k = pl.kernel(body, out_type=jax.ShapeDtypeStruct((NC, NACC, D),
                                                      jnp.float32),
                  mesh=_mesh(), scratch_types=scratch)
    return k(dstg)[:, :, :16]


ROWS = 400
GRID = N // ROWS


def _mean_plus_root(sums, cnts, h, wlt, bl, wrt):
    ssum = sums[0] + sums[1]
    cnt = cnts[0] + cnts[1]
    inv = 1.0 / jnp.maximum(cnt[:, 0:1], 1.0)
    mean = ssum * inv
    acc = jnp.dot(mean, wlt[...], preferred_element_type=jnp.float32,
                  precision=lax.Precision.HIGHEST)
    acc += jnp.dot(h[...], wrt[...], preferred_element_type=jnp.float32,
                   precision=lax.Precision.HIGHEST)
    return acc + bl[...]


def _dense_body(sums, cnts, h, wlt, bl, wrt, o):
    o[...] = jnp.tanh(_mean_plus_root(sums[...], cnts[...], h, wlt, bl, wrt))


def _sum_spec():
    return pl.BlockSpec((NC, ROWS, D), lambda i: (0, i, 0))


def _cnt_spec():
    return pl.BlockSpec((NC, ROWS, 16), lambda i: (0, i, 0))


def _tc_dense(sums, cnts, h, wlt, bl, wrt):
    blk = lambda shape: pl.BlockSpec(shape, lambda i: (i, 0))
    full = lambda shape: pl.BlockSpec(shape, lambda i: (0, 0))
    return pl.pallas_call(
        _dense_body,
        grid=(GRID,),
        in_specs=[_sum_spec(), _cnt_spec(), blk((ROWS, D)), full((D, D)),
                  full((1, D)), full((D, D))],
        out_specs=blk((ROWS, D)),
        out_shape=jax.ShapeDtypeStruct((N, D), jnp.float32),
    )(sums, cnts, h, wlt, bl, wrt)


def _final_body(sums, cnts, h, wlt, bl, wrt, batch, w1t, b1, w2t, b2,
                o, gmax):
    i = pl.program_id(0)

    @pl.when(i == 0)
    def _():
        gmax[...] = jnp.full((B, D), -jnp.inf, jnp.float32)

    h4 = jnp.tanh(_mean_plus_root(sums[...], cnts[...], h, wlt, bl, wrt))
    bb = batch[0]
    cur = gmax[...]
    parts = []
    for b in range(B):
        m = jnp.where(bb == b, h4, -jnp.inf)
        parts.append(jnp.max(m, axis=0, keepdims=True))
    gmax[...] = jnp.maximum(cur, jnp.concatenate(parts, axis=0))

    @pl.when(i == GRID - 1)
    def _():
        g = gmax[...]
        g = jnp.where(g > -3.0e38, g, 0.0)
        for _ in range(3):
            g = jnp.tanh(jnp.dot(g, w1t[...], preferred_element_type=jnp.float32,
                                 precision=lax.Precision.HIGHEST) + b1[...])
        o[...] = jnp.dot(g, w2t[...], preferred_element_type=jnp.float32,
                         precision=lax.Precision.HIGHEST) + b2[...]


def _tc_final(sums, cnts, h, wlt, bl, wrt, batchg, w1t, b1, w2t, b2):
    blk = lambda shape: pl.BlockSpec(shape, lambda i: (i, 0))
    full = lambda shape: pl.BlockSpec(shape, lambda i: (0, 0))
    return pl.pallas_call(
        _final_body,
        grid=(GRID,),
        in_specs=[_sum_spec(), _cnt_spec(), blk((ROWS, D)), full((D, D)),
                  full((1, D)), full((D, D)),
                  pl.BlockSpec((1, ROWS, 1), lambda i: (i, 0, 0)),
                  full((D, D)), full((1, D)), full((D, OUT)),
                  full((1, OUT))],
        out_specs=full((B, OUT)),
        out_shape=jax.ShapeDtypeStruct((B, OUT), jnp.float32),
        scratch_shapes=[pltpu.VMEM((B, D), jnp.float32)],
    )(sums, cnts, h, wlt, bl, wrt, batchg, w1t, b1, w2t, b2)


def kernel(x, edge_index, batch, Wl1, bl1, Wr1, Wl2, bl2, Wr2, W1, b1, W2, b2):
    src = edge_index[0]
    dst = edge_index[1]
    pad = EPAD - E
    srcg = jnp.concatenate([src, jnp.zeros((pad,), jnp.int32)]).reshape(
        NW, CPW, CH)
    dstg = jnp.concatenate([dst, jnp.full((pad,), N, jnp.int32)]).reshape(
        NW, CPW, CH)
    batchg = batch.reshape(GRID, ROWS, 1)

    wl1t, wr1t = Wl1.T, Wr1.T
    wl2t, wr2t = Wl2.T, Wr2.T
    w1t, w2t = W1.T, W2.T
    bl1r, bl2r = bl1.reshape(1, D), bl2.reshape(1, D)
    b1r, b2r = b1.reshape(1, D), b2.reshape(1, OUT)

    cnts = _sc_counts(dstg)
    sums = _sc_aggregate(x, srcg, dstg)
    h = _tc_dense(sums, cnts, x, wl1t, bl1r, wr1t)
    for _ in range(2):
        s2 = _sc_aggregate(h, srcg, dstg)
        h = _tc_dense(s2, cnts, h, wl2t, bl2r, wr2t)
    s4 = _sc_aggregate(h, srcg, dstg)
    return _tc_final(s4, cnts, h, wl2t, bl2r, wr2t,
                     batchg, w1t, b1r, w2t, b2r)


# speedup vs baseline: 3.3400x; 1.0888x over previous
import functools

import jax
import jax.numpy as jnp
from jax import lax
from jax.experimental import pallas as pl
from jax.experimental.pallas import tpu as pltpu
from jax.experimental.pallas import tpu_sc as plsc

N = 10000
E = 320000
D = 128
B = 16
OUT = 3

NC = 2
NS = 16
NW = NC * NS
CH = 128
GRP = 8
CPW = -(-E // (NW * CH * GRP)) * GRP
EPAD = NW * CH * CPW
NACC = -(-(N + 1) // (8 * NS)) * 8 * NS
RPS = NACC // NS


def _mesh():
    return plsc.VectorSubcoreMesh(core_axis_name="c", subcore_axis_name="s",
                                  num_cores=NC, num_subcores=NS)


def _zero_shared(buf_v, shared, base, width):
    def zrow(r, _):
        z = jnp.zeros((16,), jnp.float32)
        for l in range(width // 16):
            buf_v[r, pl.ds(l * 16, 16)] = z
        return 0
    lax.fori_loop(0, CH, zrow, 0)
    nfull = RPS // CH
    def zcopy(r, _):
        pltpu.sync_copy(buf_v, shared.at[pl.ds(base + r * CH, CH)])
        return 0
    lax.fori_loop(0, nfull, zcopy, 0)
    rem = RPS - nfull * CH
    if rem:
        pltpu.sync_copy(buf_v.at[pl.ds(0, rem)],
                        shared.at[pl.ds(base + nfull * CH, rem)])


def _sc_aggregate(x, srcg, dstg):
    scratch = [
        pltpu.VMEM_SHARED((NACC, D), jnp.float32),
        pltpu.VMEM((GRP, CH), jnp.int32),
        pltpu.VMEM((GRP, CH), jnp.int32),
        pltpu.VMEM((2, CH, D), jnp.float32),
        pltpu.SemaphoreType.DMA,
        pltpu.SemaphoreType.DMA,
        pltpu.SemaphoreType.DMA,
        pltpu.SemaphoreType.DMA,
    ]

    def body(x_hbm, srcg_hbm, dstg_hbm, out_hbm, acc, src_v, dst_v, rows_v,
             gs0, gs1, ss0, ss1):
        c = lax.axis_index("c")
        s = lax.axis_index("s")
        wid = s * NC + c
        base = s * RPS
        _zero_shared(rows_v.at[0], acc, base, D)
        plsc.subcore_barrier()
        gsem = [gs0, gs1]
        ssem = [ss0, ss1]

        def group(g, _):
            pltpu.sync_copy(srcg_hbm.at[wid].at[pl.ds(g * GRP, GRP)], src_v)
            pltpu.sync_copy(dstg_hbm.at[wid].at[pl.ds(g * GRP, GRP)], dst_v)
            dg = [None] * GRP
            dsc = [None] * GRP
            for j in range(GRP):
                b = j % 2
                if j >= 2:
                    dsc[j - 2].wait()
                dg[j] = pltpu.async_copy(x_hbm.at[src_v.at[j]],
                                         rows_v.at[b], gsem[b])
                if j >= 1:
                    dg[j - 1].wait()
                    dsc[j - 1] = pltpu.async_copy(
                        rows_v.at[1 - b], acc.at[dst_v.at[j - 1]],
                        ssem[1 - b], add=True)
            bl_ = (GRP - 1) % 2
            dg[GRP - 1].wait()
            dsc[GRP - 1] = pltpu.async_copy(
                rows_v.at[bl_], acc.at[dst_v.at[GRP - 1]], ssem[bl_],
                add=True)
            dsc[GRP - 2].wait()
            dsc[GRP - 1].wait()
            return 0
        lax.fori_loop(0, CPW // GRP, group, 0)

        plsc.subcore_barrier()
        pltpu.sync_copy(acc.at[pl.ds(base, RPS)],
                        out_hbm.at[c].at[pl.ds(base, RPS)])

    k = pl.kernel(body, out_type=jax.ShapeDtypeStruct((NC, NACC, D),
                                                      jnp.float32),
                  mesh=_mesh(), scratch_types=scratch)
    return k(x, srcg, dstg)


def _sc_counts(dstg):
    scratch = [
        pltpu.VMEM_SHARED((NACC, D), jnp.float32),
        pltpu.VMEM((GRP, CH), jnp.int32),
        pltpu.VMEM((CH, D), jnp.float32),
    ]

    def body(dstg_hbm, cnt_hbm, cacc, dst_v, ones_v):
        c = lax.axis_index("c")
        s = lax.axis_index("s")
        wid = s * NC + c
        base = s * RPS
        _zero_shared(ones_v, cacc, base, D)
        def orow(r, _):
            one = jnp.ones((16,), jnp.float32)
            ones_v[r, pl.ds(0, 16)] = one
            return 0
        lax.fori_loop(0, CH, orow, 0)
        plsc.subcore_barrier()

        def group(g, _):
            pltpu.sync_copy(dstg_hbm.at[wid].at[pl.ds(g * GRP, GRP)], dst_v)
            for j in range(GRP):
                pltpu.sync_copy(ones_v, cacc.at[dst_v.at[j]], add=True)
            return 0
        lax.fori_loop(0, CPW // GRP, group, 0)

        plsc.subcore_barrier()
        pltpu.sync_copy(cacc.at[pl.ds(base, RPS)],
                        cnt_hbm.at[c].at[pl.ds(base, RPS)])

    k = pl.kernel(body, out_type=jax.ShapeDtypeStruct((NC, NACC, D),
                                                      jnp.float32),
                  mesh=_mesh(), scratch_types=scratch)
    return k(dstg)[:, :, :16]


ROWS = 400
GRID = N // ROWS


def _mean_plus_root(sums, cnts, h, wlt, bl, wrt):
    ssum = sums[0] + sums[1]
    cnt = cnts[0] + cnts[1]
    inv = 1.0 / jnp.maximum(cnt[:, 0:1], 1.0)
    mean = ssum * inv
    acc = jnp.dot(mean, wlt[...], preferred_element_type=jnp.float32,
                  precision=lax.Precision.HIGHEST)
    acc += jnp.dot(h[...], wrt[...], preferred_element_type=jnp.float32,
                   precision=lax.Precision.HIGHEST)
    return acc + bl[...]


def _dense_body(sums, cnts, h, wlt, bl, wrt, o):
    o[...] = jnp.tanh(_mean_plus_root(sums[...], cnts[...], h, wlt, bl, wrt))


def _sum_spec():
    return pl.BlockSpec((NC, ROWS, D), lambda i: (0, i, 0))


def _cnt_spec():
    return pl.BlockSpec((NC, ROWS, 16), lambda i: (0, i, 0))


def _tc_dense(sums, cnts, h, wlt, bl, wrt):
    blk = lambda shape: pl.BlockSpec(shape, lambda i: (i, 0))
    full = lambda shape: pl.BlockSpec(shape, lambda i: (0, 0))
    return pl.pallas_call(
        _dense_body,
        grid=(GRID,),
        in_specs=[_sum_spec(), _cnt_spec(), blk((ROWS, D)), full((D, D)),
                  full((1, D)), full((D, D))],
        out_specs=blk((ROWS, D)),
        out_shape=jax.ShapeDtypeStruct((N, D), jnp.float32),
    )(sums, cnts, h, wlt, bl, wrt)


def _final_body(sums, cnts, h, wlt, bl, wrt, batch, w1t, b1, w2t, b2,
                o, gmax):
    i = pl.program_id(0)

    @pl.when(i == 0)
    def _():
        gmax[...] = jnp.full((B, D), -jnp.inf, jnp.float32)

    h4 = jnp.tanh(_mean_plus_root(sums[...], cnts[...], h, wlt, bl, wrt))
    bb = batch[0]
    cur = gmax[...]
    parts = []
    for b in range(B):
        m = jnp.where(bb == b, h4, -jnp.inf)
        parts.append(jnp.max(m, axis=0, keepdims=True))
    gmax[...] = jnp.maximum(cur, jnp.concatenate(parts, axis=0))

    @pl.when(i == GRID - 1)
    def _():
        g = gmax[...]
        g = jnp.where(g > -3.0e38, g, 0.0)
        for _ in range(3):
            g = jnp.tanh(jnp.dot(g, w1t[...], preferred_element_type=jnp.float32,
                                 precision=lax.Precision.HIGHEST) + b1[...])
        o[...] = jnp.dot(g, w2t[...], preferred_element_type=jnp.float32,
                         precision=lax.Precision.HIGHEST) + b2[...]


def _tc_final(sums, cnts, h, wlt, bl, wrt, batchg, w1t, b1, w2t, b2):
    blk = lambda shape: pl.BlockSpec(shape, lambda i: (i, 0))
    full = lambda shape: pl.BlockSpec(shape, lambda i: (0, 0))
    return pl.pallas_call(
        _final_body,
        grid=(GRID,),
        in_specs=[_sum_spec(), _cnt_spec(), blk((ROWS, D)), full((D, D)),
                  full((1, D)), full((D, D)),
                  pl.BlockSpec((1, ROWS, 1), lambda i: (i, 0, 0)),
                  full((D, D)), full((1, D)), full((D, OUT)),
                  full((1, OUT))],
        out_specs=full((B, OUT)),
        out_shape=jax.ShapeDtypeStruct((B, OUT), jnp.float32),
        scratch_shapes=[pltpu.VMEM((B, D), jnp.float32)],
    )(sums, cnts, h, wlt, bl, wrt, batchg, w1t, b1, w2t, b2)


def kernel(x, edge_index, batch, Wl1, bl1, Wr1, Wl2, bl2, Wr2, W1, b1, W2, b2):
    src = edge_index[0]
    dst = edge_index[1]
    pad = EPAD - E
    srcg = jnp.concatenate([src, jnp.zeros((pad,), jnp.int32)]).reshape(
        NW, CPW, CH)
    dstg = jnp.concatenate([dst, jnp.full((pad,), N, jnp.int32)]).reshape(
        NW, CPW, CH)
    batchg = batch.reshape(GRID, ROWS, 1)

    wl1t, wr1t = Wl1.T, Wr1.T
    wl2t, wr2t = Wl2.T, Wr2.T
    w1t, w2t = W1.T, W2.T
    bl1r, bl2r = bl1.reshape(1, D), bl2.reshape(1, D)
    b1r, b2r = b1.reshape(1, D), b2.reshape(1, OUT)

    cnts = _sc_counts(dstg)
    sums = _sc_aggregate(x, srcg, dstg)
    h = _tc_dense(sums, cnts, x, wl1t, bl1r, wr1t)
    for _ in range(2):
        s2 = _sc_aggregate(h, srcg, dstg)
        h = _tc_dense(s2, cnts, h, wl2t, bl2r, wr2t)
    s4 = _sc_aggregate(h, srcg, dstg)
    return _tc_final(s4, cnts, h, wl2t, bl2r, wr2t,
                     batchg, w1t, b1r, w2t, b2r)


# speedup vs baseline: 3.3414x; 1.0004x over previous
import functools

import jax
import jax.numpy as jnp
from jax import lax
from jax.experimental import pallas as pl
from jax.experimental.pallas import tpu as pltpu
from jax.experimental.pallas import tpu_sc as plsc

N = 10000
E = 320000
D = 128
B = 16
OUT = 3

NC = 2
NS = 16
NW = NC * NS
CH = 128
GRP = 8
CPW = -(-E // (NW * CH * GRP)) * GRP
EPAD = NW * CH * CPW
NACC = -(-(N + 1) // (8 * NS)) * 8 * NS
RPS = NACC // NS


def _mesh():
    return plsc.VectorSubcoreMesh(core_axis_name="c", subcore_axis_name="s",
                                  num_cores=NC, num_subcores=NS)


def _zero_shared(buf_v, shared, base, width):
    def zrow(r, _):
        z = jnp.zeros((16,), jnp.float32)
        for l in range(width // 16):
            buf_v[r, pl.ds(l * 16, 16)] = z
        return 0
    lax.fori_loop(0, CH, zrow, 0)
    nfull = RPS // CH
    def zcopy(r, _):
        pltpu.sync_copy(buf_v, shared.at[pl.ds(base + r * CH, CH)])
        return 0
    lax.fori_loop(0, nfull, zcopy, 0)
    rem = RPS - nfull * CH
    if rem:
        pltpu.sync_copy(buf_v.at[pl.ds(0, rem)],
                        shared.at[pl.ds(base + nfull * CH, rem)])


def _sc_aggregate(x, srcg, dstg):
    scratch = [
        pltpu.VMEM_SHARED((NACC, D), jnp.float32),
        pltpu.VMEM((GRP, CH), jnp.int32),
        pltpu.VMEM((GRP, CH), jnp.int32),
        pltpu.VMEM((2, CH, D), jnp.float32),
        pltpu.SemaphoreType.DMA,
        pltpu.SemaphoreType.DMA,
        pltpu.SemaphoreType.DMA,
        pltpu.SemaphoreType.DMA,
    ]

    def body(x_hbm, srcg_hbm, dstg_hbm, out_hbm, acc, src_v, dst_v, rows_v,
             gs0, gs1, ss0, ss1):
        c = lax.axis_index("c")
        s = lax.axis_index("s")
        wid = s * NC + c
        base = s * RPS
        _zero_shared(rows_v.at[0], acc, base, D)
        plsc.subcore_barrier()
        gsem = [gs0, gs1]
        ssem = [ss0, ss1]

        def group(g, _):
            pltpu.sync_copy(srcg_hbm.at[wid].at[pl.ds(g * GRP, GRP)], src_v)
            pltpu.sync_copy(dstg_hbm.at[wid].at[pl.ds(g * GRP, GRP)], dst_v)
            dg = [None] * GRP
            dsc = [None] * GRP
            for j in range(GRP):
                b = j % 2
                if j >= 2:
                    dsc[j - 2].wait()
                dg[j] = pltpu.async_copy(x_hbm.at[src_v.at[j]],
                                         rows_v.at[b], gsem[b])
                if j >= 1:
                    dg[j - 1].wait()
                    dsc[j - 1] = pltpu.async_copy(
                        rows_v.at[1 - b], acc.at[dst_v.at[j - 1]],
                        ssem[1 - b], add=True)
            bl_ = (GRP - 1) % 2
            dg[GRP - 1].wait()
            dsc[GRP - 1] = pltpu.async_copy(
                rows_v.at[bl_], acc.at[dst_v.at[GRP - 1]], ssem[bl_],
                add=True)
            dsc[GRP - 2].wait()
            dsc[GRP - 1].wait()
            return 0
        lax.fori_loop(0, CPW // GRP, group, 0)

        plsc.subcore_barrier()
        pltpu.sync_copy(acc.at[pl.ds(base, RPS)],
                        out_hbm.at[c].at[pl.ds(base, RPS)])

    k = pl.kernel(body, out_type=jax.ShapeDtypeStruct((NC, NACC, D),
                                                      jnp.float32),
                  mesh=_mesh(), scratch_types=scratch)
    return k(x, srcg, dstg)


def _sc_counts(dstg):
    scratch = [
        pltpu.VMEM_SHARED((NACC, D), jnp.float32),
        pltpu.VMEM((GRP, CH), jnp.int32),
        pltpu.VMEM((CH, D), jnp.float32),
    ]

    def body(dstg_hbm, cnt_hbm, cacc, dst_v, ones_v):
        c = lax.axis_index("c")
        s = lax.axis_index("s")
        wid = s * NC + c
        base = s * RPS
        _zero_shared(ones_v, cacc, base, D)
        def orow(r, _):
            one = jnp.ones((16,), jnp.float32)
            ones_v[r, pl.ds(0, 16)] = one
            return 0
        lax.fori_loop(0, CH, orow, 0)
        plsc.subcore_barrier()

        def group(g, _):
            pltpu.sync_copy(dstg_hbm.at[wid].at[pl.ds(g * GRP, GRP)], dst_v)
            for j in range(GRP):
                pltpu.sync_copy(ones_v, cacc.at[dst_v.at[j]], add=True)
            return 0
        lax.fori_loop(0, CPW // GRP, group, 0)

        plsc.subcore_barrier()
        pltpu.sync_copy(cacc.at[pl.ds(base, RPS)],
                        cnt_hbm.at[c].at[pl.ds(base, RPS)])

    k = pl.kernel(body, out_type=jax.ShapeDtypeStruct((NC, NACC, D),
                                                      jnp.float32),
                  mesh=_mesh(), scratch_types=scratch)
    return k(dstg)[:, :, :16]


ROWS = 400
GRID = N // ROWS


def _mean_plus_root(sums, cnts, h, wlt, bl, wrt):
    ssum = sums[0] + sums[1]
    cnt = cnts[0] + cnts[1]
    inv = 1.0 / jnp.maximum(cnt[:, 0:1], 1.0)
    mean = ssum * inv
    acc = jnp.dot(mean, wlt[...], preferred_element_type=jnp.float32,
                  precision=lax.Precision.HIGHEST)
    acc += jnp.dot(h[...], wrt[...], preferred_element_type=jnp.float32,
                   precision=lax.Precision.HIGHEST)
    return acc + bl[...]


def _dense_body(sums, cnts, h, wlt, bl, wrt, o):
    o[...] = jnp.tanh(_mean_plus_root(sums[...], cnts[...], h, wlt, bl, wrt))


def _sum_spec():
    return pl.BlockSpec((NC, ROWS, D), lambda i: (0, i, 0))


def _cnt_spec():
    return pl.BlockSpec((NC, ROWS, 16), lambda i: (0, i, 0))


def _tc_dense(sums, cnts, h, wlt, bl, wrt):
    blk = lambda shape: pl.BlockSpec(shape, lambda i: (i, 0))
    full = lambda shape: pl.BlockSpec(shape, lambda i: (0, 0))
    return pl.pallas_call(
        _dense_body,
        grid=(GRID,),
        in_specs=[_sum_spec(), _cnt_spec(), blk((ROWS, D)), full((D, D)),
                  full((1, D)), full((D, D))],
        out_specs=blk((ROWS, D)),
        out_shape=jax.ShapeDtypeStruct((N, D), jnp.float32),
    )(sums, cnts, h, wlt, bl, wrt)


def _final_body(sums, cnts, h, wlt, bl, wrt, batch, w1t, b1, w2t, b2,
                o, gmax):
    i = pl.program_id(0)

    @pl.when(i == 0)
    def _():
        gmax[...] = jnp.full((B, D), -jnp.inf, jnp.float32)

    h4 = jnp.tanh(_mean_plus_root(sums[...], cnts[...], h, wlt, bl, wrt))
    bb = batch[0]
    cur = gmax[...]
    parts = []
    for b in range(B):
        m = jnp.where(bb == b, h4, -jnp.inf)
        parts.append(jnp.max(m, axis=0, keepdims=True))
    gmax[...] = jnp.maximum(cur, jnp.concatenate(parts, axis=0))

    @pl.when(i == GRID - 1)
    def _():
        g = gmax[...]
        g = jnp.where(g > -3.0e38, g, 0.0)
        for _ in range(3):
            g = jnp.tanh(jnp.dot(g, w1t[...], preferred_element_type=jnp.float32,
                                 precision=lax.Precision.HIGHEST) + b1[...])
        o[...] = jnp.dot(g, w2t[...], preferred_element_type=jnp.float32,
                         precision=lax.Precision.HIGHEST) + b2[...]


def _tc_final(sums, cnts, h, wlt, bl, wrt, batchg, w1t, b1, w2t, b2):
    blk = lambda shape: pl.BlockSpec(shape, lambda i: (i, 0))
    full = lambda shape: pl.BlockSpec(shape, lambda i: (0, 0))
    return pl.pallas_call(
        _final_body,
        grid=(GRID,),
        in_specs=[_sum_spec(), _cnt_spec(), blk((ROWS, D)), full((D, D)),
                  full((1, D)), full((D, D)),
                  pl.BlockSpec((1, ROWS, 1), lambda i: (i, 0, 0)),
                  full((D, D)), full((1, D)), full((D, OUT)),
                  full((1, OUT))],
        out_specs=full((B, OUT)),
        out_shape=jax.ShapeDtypeStruct((B, OUT), jnp.float32),
        scratch_shapes=[pltpu.VMEM((B, D), jnp.float32)],
    )(sums, cnts, h, wlt, bl, wrt, batchg, w1t, b1, w2t, b2)


def kernel(x, edge_index, batch, Wl1, bl1, Wr1, Wl2, bl2, Wr2, W1, b1, W2, b2):
    src = edge_index[0]
    dst = edge_index[1]
    pad = EPAD - E
    srcg = jnp.concatenate([src, jnp.zeros((pad,), jnp.int32)]).reshape(
        NW, CPW, CH)
    trash = N + jnp.arange(pad, dtype=jnp.int32) % (NACC - N)
    dstg = jnp.concatenate([dst, trash]).reshape(NW, CPW, CH)
    batchg = batch.reshape(GRID, ROWS, 1)

    wl1t, wr1t = Wl1.T, Wr1.T
    wl2t, wr2t = Wl2.T, Wr2.T
    w1t, w2t = W1.T, W2.T
    bl1r, bl2r = bl1.reshape(1, D), bl2.reshape(1, D)
    b1r, b2r = b1.reshape(1, D), b2.reshape(1, OUT)

    cnts = _sc_counts(dstg)
    sums = _sc_aggregate(x, srcg, dstg)
    h = _tc_dense(sums, cnts, x, wl1t, bl1r, wr1t)
    for _ in range(2):
        s2 = _sc_aggregate(h, srcg, dstg)
        h = _tc_dense(s2, cnts, h, wl2t, bl2r, wr2t)
    s4 = _sc_aggregate(h, srcg, dstg)
    return _tc_final(s4, cnts, h, wl2t, bl2r, wr2t,
                     batchg, w1t, b1r, w2t, b2r)


# speedup vs baseline: 3.3423x; 1.0003x over previous
import functools

import jax
import jax.numpy as jnp
from jax import lax
from jax.experimental import pallas as pl
from jax.experimental.pallas import tpu as pltpu
from jax.experimental.pallas import tpu_sc as plsc

N = 10000
E = 320000
D = 128
B = 16
OUT = 3

NC = 2
NS = 16
NW = NC * NS
CH = 128
GRP = 8
CPW = -(-E // (NW * CH * GRP)) * GRP
EPAD = NW * CH * CPW
NACC = -(-(N + 1) // (8 * NS)) * 8 * NS
RPS = NACC // NS


def _mesh():
    return plsc.VectorSubcoreMesh(core_axis_name="c", subcore_axis_name="s",
                                  num_cores=NC, num_subcores=NS)


def _zero_shared(buf_v, shared, base, width):
    def zrow(r, _):
        z = jnp.zeros((16,), jnp.float32)
        for l in range(width // 16):
            buf_v[r, pl.ds(l * 16, 16)] = z
        return 0
    lax.fori_loop(0, CH, zrow, 0)
    nfull = RPS // CH
    def zcopy(r, _):
        pltpu.sync_copy(buf_v, shared.at[pl.ds(base + r * CH, CH)])
        return 0
    lax.fori_loop(0, nfull, zcopy, 0)
    rem = RPS - nfull * CH
    if rem:
        pltpu.sync_copy(buf_v.at[pl.ds(0, rem)],
                        shared.at[pl.ds(base + nfull * CH, rem)])


def _sc_aggregate(x, srcg, dstg):
    scratch = [
        pltpu.VMEM_SHARED((NACC, D), jnp.float32),
        pltpu.VMEM((GRP, CH), jnp.int32),
        pltpu.VMEM((GRP, CH), jnp.int32),
        pltpu.VMEM((2, CH, D), jnp.float32),
        pltpu.SemaphoreType.DMA,
        pltpu.SemaphoreType.DMA,
        pltpu.SemaphoreType.DMA,
        pltpu.SemaphoreType.DMA,
    ]

    def body(x_hbm, srcg_hbm, dstg_hbm, out_hbm, acc, src_v, dst_v, rows_v,
             gs0, gs1, ss0, ss1):
        c = lax.axis_index("c")
        s = lax.axis_index("s")
        wid = s * NC + c
        base = s * RPS
        _zero_shared(rows_v.at[0], acc, base, D)
        plsc.subcore_barrier()
        gsem = [gs0, gs1]
        ssem = [ss0, ss1]

        def group(g, _):
            pltpu.sync_copy(srcg_hbm.at[wid].at[pl.ds(g * GRP, GRP)], src_v)
            pltpu.sync_copy(dstg_hbm.at[wid].at[pl.ds(g * GRP, GRP)], dst_v)
            dg = [None] * GRP
            dsc = [None] * GRP
            for j in range(GRP):
                b = j % 2
                if j >= 2:
                    dsc[j - 2].wait()
                dg[j] = pltpu.async_copy(x_hbm.at[src_v.at[j]],
                                         rows_v.at[b], gsem[b])
                if j >= 1:
                    dg[j - 1].wait()
                    dsc[j - 1] = pltpu.async_copy(
                        rows_v.at[1 - b], acc.at[dst_v.at[j - 1]],
                        ssem[1 - b], add=True)
            bl_ = (GRP - 1) % 2
            dg[GRP - 1].wait()
            dsc[GRP - 1] = pltpu.async_copy(
                rows_v.at[bl_], acc.at[dst_v.at[GRP - 1]], ssem[bl_],
                add=True)
            dsc[GRP - 2].wait()
            dsc[GRP - 1].wait()
            return 0
        lax.fori_loop(0, CPW // GRP, group, 0)

        plsc.subcore_barrier()
        pltpu.sync_copy(acc.at[pl.ds(base, RPS)],
                        out_hbm.at[c].at[pl.ds(base, RPS)])

    k = pl.kernel(body, out_type=jax.ShapeDtypeStruct((NC, NACC, D),
                                                      jnp.float32),
                  mesh=_mesh(), scratch_types=scratch)
    return k(x, srcg, dstg)


def _sc_counts(dstg):
    scratch = [
        pltpu.VMEM_SHARED((NACC, D), jnp.float32),
        pltpu.VMEM((GRP, CH), jnp.int32),
        pltpu.VMEM((CH, D), jnp.float32),
    ]

    def body(dstg_hbm, cnt_hbm, cacc, dst_v, ones_v):
        c = lax.axis_index("c")
        s = lax.axis_index("s")
        wid = s * NC + c
        base = s * RPS
        _zero_shared(ones_v, cacc, base, D)
        def orow(r, _):
            one = jnp.ones((16,), jnp.float32)
            ones_v[r, pl.ds(0, 16)] = one
            return 0
        lax.fori_loop(0, CH, orow, 0)
        plsc.subcore_barrier()

        def group(g, _):
            pltpu.sync_copy(dstg_hbm.at[wid].at[pl.ds(g * GRP, GRP)], dst_v)
            for j in range(GRP):
                pltpu.sync_copy(ones_v, cacc.at[dst_v.at[j]], add=True)
            return 0
        lax.fori_loop(0, CPW // GRP, group, 0)

        plsc.subcore_barrier()
        pltpu.sync_copy(cacc.at[pl.ds(base, RPS)],
                        cnt_hbm.at[c].at[pl.ds(base, RPS)])

    k = pl.kernel(body, out_type=jax.ShapeDtypeStruct((NC, NACC, D),
                                                      jnp.float32),
                  mesh=_mesh(), scratch_types=scratch)
    return k(dstg)[:, :, :16]


ROWS = 400
GRID = N // ROWS


def _mean_plus_root(sums, cnts, h, wlt, bl, wrt):
    ssum = sums[0] + sums[1]
    cnt = cnts[0] + cnts[1]
    inv = 1.0 / jnp.maximum(cnt[:, 0:1], 1.0)
    mean = ssum * inv
    acc = jnp.dot(mean, wlt[...], preferred_element_type=jnp.float32,
                  precision=lax.Precision.HIGHEST)
    acc += jnp.dot(h[...], wrt[...], preferred_element_type=jnp.float32,
                   precision=lax.Precision.HIGHEST)
    return acc + bl[...]


def _dense_body(sums, cnts, h, wlt, bl, wrt, o):
    o[...] = jnp.tanh(_mean_plus_root(sums[...], cnts[...], h, wlt, bl, wrt))


def _sum_spec():
    return pl.BlockSpec((NC, ROWS, D), lambda i: (0, i, 0))


def _cnt_spec():
    return pl.BlockSpec((NC, ROWS, 16), lambda i: (0, i, 0))


def _tc_dense(sums, cnts, h, wlt, bl, wrt):
    blk = lambda shape: pl.BlockSpec(shape, lambda i: (i, 0))
    full = lambda shape: pl.BlockSpec(shape, lambda i: (0, 0))
    return pl.pallas_call(
        _dense_body,
        grid=(GRID,),
        in_specs=[_sum_spec(), _cnt_spec(), blk((ROWS, D)), full((D, D)),
                  full((1, D)), full((D, D))],
        out_specs=blk((ROWS, D)),
        out_shape=jax.ShapeDtypeStruct((N, D), jnp.float32),
    )(sums, cnts, h, wlt, bl, wrt)


def _final_body(sums, cnts, h, wlt, bl, wrt, batch, w1t, b1, w2t, b2,
                o, gmax):
    i = pl.program_id(0)

    @pl.when(i == 0)
    def _():
        gmax[...] = jnp.full((B, D), -jnp.inf, jnp.float32)

    h4 = jnp.tanh(_mean_plus_root(sums[...], cnts[...], h, wlt, bl, wrt))
    bb = batch[0]
    cur = gmax[...]
    parts = []
    for b in range(B):
        m = jnp.where(bb == b, h4, -jnp.inf)
        parts.append(jnp.max(m, axis=0, keepdims=True))
    gmax[...] = jnp.maximum(cur, jnp.concatenate(parts, axis=0))

    @pl.when(i == GRID - 1)
    def _():
        g = gmax[...]
        g = jnp.where(g > -3.0e38, g, 0.0)
        for _ in range(3):
            g = jnp.tanh(jnp.dot(g, w1t[...], preferred_element_type=jnp.float32,
                                 precision=lax.Precision.HIGHEST) + b1[...])
        o[...] = jnp.dot(g, w2t[...], preferred_element_type=jnp.float32,
                         precision=lax.Precision.HIGHEST) + b2[...]


def _tc_final(sums, cnts, h, wlt, bl, wrt, batchg, w1t, b1, w2t, b2):
    blk = lambda shape: pl.BlockSpec(shape, lambda i: (i, 0))
    full = lambda shape: pl.BlockSpec(shape, lambda i: (0, 0))
    return pl.pallas_call(
        _final_body,
        grid=(GRID,),
        in_specs=[_sum_spec(), _cnt_spec(), blk((ROWS, D)), full((D, D)),
                  full((1, D)), full((D, D)),
                  pl.BlockSpec((1, ROWS, 1), lambda i: (i, 0, 0)),
                  full((D, D)), full((1, D)), full((D, OUT)),
                  full((1, OUT))],
        out_specs=full((B, OUT)),
        out_shape=jax.ShapeDtypeStruct((B, OUT), jnp.float32),
        scratch_shapes=[pltpu.VMEM((B, D), jnp.float32)],
    )(sums, cnts, h, wlt, bl, wrt, batchg, w1t, b1, w2t, b2)


def kernel(x, edge_index, batch, Wl1, bl1, Wr1, Wl2, bl2, Wr2, W1, b1, W2, b2):
    src = edge_index[0]
    dst = edge_index[1]
    pad = EPAD - E
    srcp = jnp.concatenate([src, jnp.zeros((pad,), jnp.int32)])
    trash = N + jnp.arange(pad, dtype=jnp.int32) % (NACC - N)
    dstp = jnp.concatenate([dst, trash])
    srcg = srcp.reshape(NW, CPW, CH)
    dstg = dstp.reshape(NW, CPW, CH)
    batchg = batch.reshape(GRID, ROWS, 1)

    wl1t, wr1t = Wl1.T, Wr1.T
    wl2t, wr2t = Wl2.T, Wr2.T
    w1t, w2t = W1.T, W2.T
    bl1r, bl2r = bl1.reshape(1, D), bl2.reshape(1, D)
    b1r, b2r = b1.reshape(1, D), b2.reshape(1, OUT)

    cnts = _sc_counts(dstg)
    sums = _sc_aggregate(x, srcg, dstg)
    h = _tc_dense(sums, cnts, x, wl1t, bl1r, wr1t)
    for _ in range(2):
        s2 = _sc_aggregate(h, srcg, dstg)
        h = _tc_dense(s2, cnts, h, wl2t, bl2r, wr2t)
    s4 = _sc_aggregate(h, srcg, dstg)
    return _tc_final(s4, cnts, h, wl2t, bl2r, wr2t,
                     batchg, w1t, b1r, w2t, b2r)
